# baseline (speedup 1.0000x reference)
import jax
import jax.numpy as jnp
from jax import lax
from jax.experimental import pallas as pl
from jax.experimental.pallas import tpu as pltpu
from jax.experimental.pallas import tpu_sc as plsc

ROWS = 128
COLS = 32768
L = 16
VPI = 8
CHUNK = L * VPI
ITERS = COLS // CHUNK
NC = 2
NS = 16
NW = NC * NS
RPW = ROWS // NW

_INT_MAX = 2**31 - 1


def _row_argmax(buf, parity):
    neg = jnp.full((L,), -jnp.inf, jnp.float32)
    zero = jnp.zeros((L,), jnp.int32)
    init = (tuple(neg for _ in range(VPI)), tuple(zero for _ in range(VPI)))

    def body(i, carry):
        ms, mis = carry
        ivec = jnp.full((L,), i, jnp.int32)
        base = i * CHUNK
        new_ms, new_mis = [], []
        for k in range(VPI):
            v = buf[parity, pl.ds(base + k * L, L)]
            cond = v > ms[k]
            new_ms.append(jnp.where(cond, v, ms[k]))
            new_mis.append(jnp.where(cond, ivec, mis[k]))
        return (tuple(new_ms), tuple(new_mis))

    ms, mis = lax.fori_loop(0, ITERS, body, init)

    lane = lax.iota(jnp.int32, L)
    cands = []
    for k in range(VPI):
        idx = mis[k] * CHUNK + (lane + (k * L))
        cands.append((ms[k], idx))

    while len(cands) > 1:
        nxt = []
        for j in range(0, len(cands), 2):
            (va, ia), (vb, ib) = cands[j], cands[j + 1]
            take_b = (vb > va) | ((vb == va) & (ib < ia))
            nxt.append((jnp.where(take_b, vb, va), jnp.where(take_b, ib, ia)))
        cands = nxt
    m, idx = cands[0]

    gmax = jnp.max(m)
    masked = jnp.where(m == gmax, idx, jnp.int32(_INT_MAX))
    return jnp.min(masked)


def _sc_body(x_hbm, out_hbm, buf, ans, sem):
    wid = lax.axis_index("s") * NC + lax.axis_index("c")
    row0 = wid * RPW
    lane = lax.iota(jnp.int32, L)

    pltpu.async_copy(x_hbm.at[row0], buf.at[0], sem)

    def row_body(r, acc):
        @pl.when(r < RPW - 1)
        def _():
            pltpu.async_copy(x_hbm.at[row0 + r + 1], buf.at[(r + 1) & 1], sem)

        pltpu.make_async_copy(x_hbm.at[row0], buf.at[0], sem).wait()
        a = _row_argmax(buf, r & 1)
        return jnp.where(lane == r, jnp.full((L,), a, jnp.int32), acc)

    acc = lax.fori_loop(0, RPW, row_body, jnp.zeros((L,), jnp.int32))
    ans[...] = acc
    pltpu.sync_copy(ans, out_hbm.at[wid])


@jax.jit
def kernel(x):
    mesh = plsc.VectorSubcoreMesh(core_axis_name="c", subcore_axis_name="s")
    out = pl.kernel(
        _sc_body,
        mesh=mesh,
        out_type=jax.ShapeDtypeStruct((NW, L), jnp.int32),
        scratch_types=[
            pltpu.VMEM((2, COLS), jnp.float32),
            pltpu.VMEM((L,), jnp.int32),
            pltpu.SemaphoreType.DMA,
        ],
        compiler_params=pltpu.CompilerParams(needs_layout_passes=False),
    )(x)
    return out[:, :RPW].reshape(ROWS).astype(jnp.int64)


# speedup vs baseline: 2.2576x; 2.2576x over previous
import jax
import jax.numpy as jnp
from jax import lax
from jax.experimental import pallas as pl
from jax.experimental.pallas import tpu as pltpu

ROWS = 128
COLS = 32768
BR = 16
GRID = ROWS // BR
_INT_MAX = 2**31 - 1


def _body(x_ref, o_ref):
    xb = x_ref[...]
    m = jnp.max(xb, axis=1, keepdims=True)
    iota = lax.broadcasted_iota(jnp.int32, (BR, COLS), 1)
    masked = jnp.where(xb == m, iota, jnp.int32(_INT_MAX))
    o_ref[0, 0, :] = jnp.min(masked, axis=1)


@jax.jit
def kernel(x):
    out = pl.pallas_call(
        _body,
        grid=(GRID,),
        in_specs=[pl.BlockSpec((BR, COLS), lambda i: (i, 0))],
        out_specs=pl.BlockSpec((1, 1, BR), lambda i: (i, 0, 0)),
        out_shape=jax.ShapeDtypeStruct((GRID, 1, BR), jnp.int32),
        compiler_params=pltpu.CompilerParams(
            dimension_semantics=("arbitrary",),
        ),
    )(x)
    return out.reshape(ROWS).astype(jnp.int64)


# speedup vs baseline: 2.7680x; 1.2261x over previous
import jax
import jax.numpy as jnp
from jax import lax
from jax.experimental import pallas as pl
from jax.experimental.pallas import tpu as pltpu

ROWS = 128
COLS = 32768
BR = 16
NBLK = ROWS // BR
NBUF = 4
_BIG = 1e9


def _blk_argmax(xb):
    m = jnp.max(xb, axis=1, keepdims=True)
    iota = lax.broadcasted_iota(jnp.int32, (BR, COLS), 1).astype(jnp.float32)
    masked = jnp.where(xb == m, iota, jnp.full((), _BIG, jnp.float32))
    return jnp.min(masked, axis=1).astype(jnp.int32)


def _body(x_hbm, o_ref, buf, sems):
    def copy(b):
        return pltpu.make_async_copy(
            x_hbm.at[pl.ds(b * BR, BR), :], buf.at[b % NBUF], sems.at[b % NBUF])

    for b in range(NBUF):
        copy(b).start()
    for b in range(NBLK):
        copy(b).wait()
        o_ref[b, 0, :] = _blk_argmax(buf[b % NBUF])
        if b + NBUF < NBLK:
            copy(b + NBUF).start()


@jax.jit
def kernel(x):
    out = pl.pallas_call(
        _body,
        in_specs=[pl.BlockSpec(memory_space=pltpu.MemorySpace.HBM)],
        out_specs=pl.BlockSpec(memory_space=pltpu.MemorySpace.VMEM),
        out_shape=jax.ShapeDtypeStruct((NBLK, 1, BR), jnp.int32),
        scratch_shapes=[
            pltpu.VMEM((NBUF, BR, COLS), jnp.float32),
            pltpu.SemaphoreType.DMA((NBUF,)),
        ],
    )(x)
    return out.reshape(ROWS).astype(jnp.int64)
